# baseline (speedup 1.0000x reference)
import jax
import jax.numpy as jnp
from jax import lax
from jax.experimental import pallas as pl
from jax.experimental.pallas import tpu as pltpu
import jax.experimental.pallas.tpu_sc as plsc

N = 10000
K = 32
DIM = 128
ATTN = 32
VAL = 32


_P1 = 2000


def _qkv_body(x_ref, wq_ref, bq_ref, wkv_ref, bkv_ref, q_ref, kv_ref):
    x = x_ref[...]
    mu = jnp.mean(x, axis=-1, keepdims=True)
    xc = x - mu
    var = jnp.mean(xc * xc, axis=-1, keepdims=True)
    h = xc * lax.rsqrt(var + 1e-5)
    q_ref[...] = jnp.dot(h, wq_ref[...], preferred_element_type=jnp.float32) + bq_ref[...]
    kv = jnp.dot(h, wkv_ref[...], preferred_element_type=jnp.float32) + bkv_ref[...]
    kv_ref[...] = kv.astype(jnp.bfloat16)


def _qkv(x2d, wq, bq, wkv, bkv):
    return pl.pallas_call(
        _qkv_body,
        grid=(N // _P1,),
        in_specs=[
            pl.BlockSpec((_P1, DIM), lambda i: (i, 0)),
            pl.BlockSpec((DIM, ATTN), lambda i: (0, 0)),
            pl.BlockSpec((1, ATTN), lambda i: (0, 0)),
            pl.BlockSpec((DIM, 2 * ATTN), lambda i: (0, 0)),
            pl.BlockSpec((1, 2 * ATTN), lambda i: (0, 0)),
        ],
        out_specs=[
            pl.BlockSpec((_P1, ATTN), lambda i: (i, 0)),
            pl.BlockSpec((_P1, 2 * ATTN), lambda i: (i, 0)),
        ],
        out_shape=[
            jax.ShapeDtypeStruct((N, ATTN), jnp.float32),
            jax.ShapeDtypeStruct((N, 2 * ATTN), jnp.bfloat16),
        ],
    )(x2d, wq, bq, wkv, bkv)


_NC = 2
_NS = 16
_NW = _NC * _NS
_E_TOT = N * K
_NPAIR = _E_TOT // 2
_CHUNK = 200


def _make_gather_body(p_per_w):
    nsteps = p_per_w // _CHUNK

    def _gather_body(kvtab, idxe_hbm, idxo_hbm, kvg_hbm,
                     idx_ve0, idx_vo0, idx_ve1, idx_vo1,
                     buf_e0, buf_o0, buf_e1, buf_o1, sem0, sem1):
        wid = lax.axis_index("s") * _NC + lax.axis_index("c")
        base = wid * p_per_w
        idx_v = ((idx_ve0, idx_vo0), (idx_ve1, idx_vo1))
        bufs = ((buf_e0, buf_o0), (buf_e1, buf_o1))
        sems = (sem0, sem1)

        def load_and_fire(i):
            s = i % 2
            off = base + i * _CHUNK
            pltpu.sync_copy(idxe_hbm.at[pl.ds(off, _CHUNK)], idx_v[s][0])
            pltpu.sync_copy(idxo_hbm.at[pl.ds(off, _CHUNK)], idx_v[s][1])
            ce = pltpu.async_copy(kvtab.at[idx_v[s][0]], bufs[s][0], sems[s])
            co = pltpu.async_copy(kvtab.at[idx_v[s][1]], bufs[s][1], sems[s])
            return ce, co

        def drain(i, descs):
            s = i % 2
            off = base + i * _CHUNK
            descs[0].wait()
            descs[1].wait()
            pltpu.sync_copy(bufs[s][0], kvg_hbm.at[pl.ds(off, _CHUNK), pl.ds(0, 2 * ATTN)])
            pltpu.sync_copy(bufs[s][1], kvg_hbm.at[pl.ds(off, _CHUNK), pl.ds(2 * ATTN, 2 * ATTN)])

        inflight = load_and_fire(0)
        for i in range(nsteps):
            nxt = load_and_fire(i + 1) if i + 1 < nsteps else None
            drain(i, inflight)
            inflight = nxt

    return _gather_body


def _sc_gather(kvtab, idx_e, idx_o):
    npair = idx_e.shape[0]
    mesh = plsc.VectorSubcoreMesh(core_axis_name="c", subcore_axis_name="s",
                                  num_cores=_NC, num_subcores=_NS)
    fn = pl.kernel(
        _make_gather_body(npair // _NW),
        out_type=jax.ShapeDtypeStruct((npair, DIM), jnp.bfloat16),
        mesh=mesh,
        compiler_params=pltpu.CompilerParams(use_tc_tiling_on_sc=False),
        scratch_types=[
            pltpu.VMEM((_CHUNK,), jnp.int32),
            pltpu.VMEM((_CHUNK,), jnp.int32),
            pltpu.VMEM((_CHUNK,), jnp.int32),
            pltpu.VMEM((_CHUNK,), jnp.int32),
            pltpu.VMEM((_CHUNK, 2 * ATTN), jnp.bfloat16),
            pltpu.VMEM((_CHUNK, 2 * ATTN), jnp.bfloat16),
            pltpu.VMEM((_CHUNK, 2 * ATTN), jnp.bfloat16),
            pltpu.VMEM((_CHUNK, 2 * ATTN), jnp.bfloat16),
            pltpu.SemaphoreType.DMA,
            pltpu.SemaphoreType.DMA,
        ],
    )
    return fn(kvtab, idx_e, idx_o)


_P3 = 400
_R3 = _P3 * K // 2
_RPP = K // 2
_HALVES = (5200, 4800)


def _block_body(x_ref, q_ref, kvg_ref, rp_ref,
                w1c, b1c, w2c, b2c, tq, wsm1, bsm1, msc, bsc, maskv,
                wo2, bo, ln2g, ln2b, f1w, f1b, f2w, f2b,
                out_ref):
    dn = (((0,), (0,)), ((), ()))
    w1 = w1c[...]
    hid = jnp.concatenate(
        [lax.dot_general(rp_ref[0, j], w1, dn, preferred_element_type=jnp.float32)[None]
         for j in range(_RPP)], axis=0)
    hid = jnp.maximum(hid.reshape(_R3, DIM) + b1c[...], 0.0)
    rarv = jnp.dot(hid, w2c[...], preferred_element_type=jnp.float32) + b2c[...]

    qrow = jnp.dot(q_ref[...], tq[...], preferred_element_type=jnp.float32)
    qe = jnp.broadcast_to(qrow[None], (_RPP, _P3, DIM)).reshape(_R3, DIM)

    kvg = kvg_ref[...].reshape(_R3, DIM).astype(jnp.float32)
    u = jnp.tanh(qe - kvg + rarv)
    s1 = jnp.maximum(jnp.dot(u, wsm1[...], preferred_element_type=jnp.float32) + bsm1[...], 0.0)
    sc = jnp.dot(s1, msc[...], preferred_element_type=jnp.float32) + bsc[...]
    sc3 = sc.reshape(_RPP, _P3, DIM)
    mask = maskv[...]

    m = jnp.max(sc3, axis=0)
    mm = jnp.maximum(m, pltpu.roll(m, 64, axis=1))
    e = jnp.exp(sc3 - mm[None]) * mask[None]
    d = jnp.sum(e, axis=0)
    d2 = d + pltpu.roll(d, 64, axis=1) + (1.0 - mask)
    attn = e / d2[None]

    vpr = (kvg + rarv).reshape(_RPP, _P3, DIM)
    vsum = jnp.sum(attn * vpr, axis=0)

    x2 = x_ref[...] + jnp.dot(vsum, wo2[...], preferred_element_type=jnp.float32) + bo[...]

    mu = jnp.mean(x2, axis=-1, keepdims=True)
    xc = x2 - mu
    var = jnp.mean(xc * xc, axis=-1, keepdims=True)
    h2 = xc * lax.rsqrt(var + 1e-5) * ln2g[...] + ln2b[...]

    f1 = jnp.dot(h2, f1w[...], preferred_element_type=jnp.float32) + f1b[...]
    g1 = f1 * 0.5 * (1.0 + lax.erf(f1 * (2.0 ** -0.5)))
    out_ref[...] = x2 + jnp.dot(g1, f2w[...], preferred_element_type=jnp.float32) + f2b[...]


def _attn_ffn(x2d, q_all, kvg, rp6, consts, nh, boff):
    def fixed(shape):
        nd = len(shape)
        return pl.BlockSpec(shape, lambda i, _nd=nd: (0,) * _nd)

    in_specs = [
        pl.BlockSpec((_P3, DIM), lambda i: (i + boff, 0)),
        pl.BlockSpec((_P3, ATTN), lambda i: (i + boff, 0)),
        pl.BlockSpec((_RPP, _P3, DIM), lambda i: (0, i, 0)),
        pl.BlockSpec((1, _RPP, 6, _P3), lambda i: (i, 0, 0, 0)),
    ] + [fixed(c.shape) for c in consts]
    return pl.pallas_call(
        _block_body,
        grid=(nh // _P3,),
        in_specs=in_specs,
        out_specs=pl.BlockSpec((_P3, DIM), lambda i: (i, 0)),
        out_shape=jax.ShapeDtypeStruct((nh, DIM), jnp.float32),
    )(x2d, q_all, kvg, rp6, *consts)


def kernel(x, knn_idx, knn_rel_pos, ln1_g, ln1_b, ln2_g, ln2_b, Wq, bq, Wk, bk,
           Wv, bv, Wo, bo, pa1_W, pa1_b, pa2_W, pa2_b, pv1_W, pv1_b, pv2_W,
           pv2_b, sm1_W, sm1_b, sm2_W, sm2_b, ffn1_W, ffn1_b, ffn2_W, ffn2_b):
    f32 = jnp.float32
    x2d = x[0]
    wq = ln1_g[:, None] * Wq
    bq2 = (bq + ln1_b @ Wq)[None, :]
    wkv_raw = jnp.concatenate([Wk, Wv], axis=1)
    wkv = ln1_g[:, None] * wkv_raw
    bkv2 = (jnp.concatenate([bk, bv]) + ln1_b @ wkv_raw)[None, :]
    q_all, kv_all = _qkv(x2d, wq, bq2, wkv, bkv2)

    idx3 = knn_idx.reshape(N, K // 2, 2).astype(jnp.int32)
    rp_t = jnp.transpose(knn_rel_pos, (2, 1, 0))
    halves = []
    p0 = 0
    for nh in _HALVES:
        idx_e = idx3[p0:p0 + nh, :, 0].transpose(1, 0).reshape(-1)
        idx_o = idx3[p0:p0 + nh, :, 1].transpose(1, 0).reshape(-1)
        nb = nh // _P3
        rp6 = (rp_t[:, :, p0:p0 + nh].reshape(3, K // 2, 2, nb, _P3)
               .transpose(3, 1, 2, 0, 4).reshape(nb, K // 2, 6, _P3))
        halves.append((nh, p0 // _P3, idx_e, idx_o, rp6))
        p0 += nh
    kvgs = [_sc_gather(kv_all, h[2], h[3]).reshape(K // 2, h[0], DIM)
            for h in halves]

    Z = jnp.zeros((ATTN, ATTN), f32)
    inv = 1.0 / jnp.sqrt(jnp.float32(ATTN))
    sm2r = jnp.broadcast_to(sm2_W * inv, (ATTN, ATTN))

    def four(b00, b01, b10, b11, b20, b21, b30, b31):
        top = jnp.concatenate([b00, b01, b10, b11], axis=1)
        bot = jnp.concatenate([b20, b21, b30, b31], axis=1)
        return top, bot

    w1c_top = jnp.concatenate([pa1_W, pv1_W, jnp.zeros((3, 2 * ATTN), f32)], axis=1)
    w1c_bot = jnp.concatenate([jnp.zeros((3, 2 * ATTN), f32), pa1_W, pv1_W], axis=1)
    w1c = jnp.concatenate([w1c_top, w1c_bot], axis=0)
    b1c = jnp.tile(jnp.concatenate([pa1_b, pv1_b]), 2)[None, :]

    r0, r1 = four(pa2_W, Z, Z, Z, Z, pv2_W, Z, Z)
    r2, r3 = four(Z, Z, pa2_W, Z, Z, Z, Z, pv2_W)
    w2c = jnp.concatenate([r0, r1, r2, r3], axis=0)
    b2c = jnp.tile(jnp.concatenate([pa2_b, pv2_b]), 2)[None, :]

    I = jnp.eye(ATTN, dtype=f32)
    tq = jnp.concatenate([I, Z, I, Z], axis=1)

    r0, r1 = four(sm1_W, Z, Z, Z, Z, Z, Z, Z)
    r2, r3 = four(Z, Z, sm1_W, Z, Z, Z, Z, Z)
    wsm1 = jnp.concatenate([r0, r1, r2, r3], axis=0)
    bsm1 = jnp.tile(jnp.concatenate([sm1_b, jnp.zeros((ATTN,), f32)]), 2)[None, :]

    r0, r1 = four(Z, sm2r, Z, Z, Z, Z, Z, Z)
    r2, r3 = four(Z, Z, Z, Z, Z, Z, Z, sm2r)
    msc = jnp.concatenate([r0, r1, r2, r3], axis=0)
    zb = jnp.zeros((ATTN,), f32)
    bsc = jnp.tile(jnp.concatenate([zb, jnp.full((ATTN,), sm2_b[0] * inv, f32)]), 2)[None, :]

    maskv = jnp.tile(jnp.concatenate([zb, jnp.ones((ATTN,), f32)]), 2)[None, :]
    wo2 = jnp.concatenate([jnp.zeros((ATTN, DIM), f32), Wo] * 2, axis=0)

    consts = [
        w1c, b1c, w2c, b2c, tq, wsm1, bsm1, msc, bsc, maskv,
        wo2, bo[None, :], ln2_g[None, :], ln2_b[None, :],
        ffn1_W, ffn1_b[None, :], ffn2_W, ffn2_b[None, :],
    ]
    outs = [_attn_ffn(x2d, q_all, kvg, h[4], consts, h[0], h[1])
            for h, kvg in zip(halves, kvgs)]
    return jnp.concatenate(outs, axis=0)[None]


# speedup vs baseline: 1.8287x; 1.8287x over previous
import jax
import jax.numpy as jnp
from jax import lax
from jax.experimental import pallas as pl
from jax.experimental.pallas import tpu as pltpu
import jax.experimental.pallas.tpu_sc as plsc

N = 10000
K = 32
DIM = 128
ATTN = 32
VAL = 32


_P1 = 2000


def _qkv_body(x_ref, wq_ref, bq_ref, wkv_ref, bkv_ref, q_ref, kv_ref):
    x = x_ref[...]
    mu = jnp.mean(x, axis=-1, keepdims=True)
    xc = x - mu
    var = jnp.mean(xc * xc, axis=-1, keepdims=True)
    h = xc * lax.rsqrt(var + 1e-5)
    q_ref[...] = jnp.dot(h, wq_ref[...], preferred_element_type=jnp.float32) + bq_ref[...]
    kv_ref[...] = jnp.dot(h, wkv_ref[...], preferred_element_type=jnp.float32) + bkv_ref[...]


def _qkv(x2d, wq, bq, wkv, bkv):
    return pl.pallas_call(
        _qkv_body,
        grid=(N // _P1,),
        in_specs=[
            pl.BlockSpec((_P1, DIM), lambda i: (i, 0)),
            pl.BlockSpec((DIM, ATTN), lambda i: (0, 0)),
            pl.BlockSpec((1, ATTN), lambda i: (0, 0)),
            pl.BlockSpec((DIM, 2 * ATTN), lambda i: (0, 0)),
            pl.BlockSpec((1, 2 * ATTN), lambda i: (0, 0)),
        ],
        out_specs=[
            pl.BlockSpec((_P1, ATTN), lambda i: (i, 0)),
            pl.BlockSpec((_P1, 2 * ATTN), lambda i: (i, 0)),
        ],
        out_shape=[
            jax.ShapeDtypeStruct((N, ATTN), jnp.float32),
            jax.ShapeDtypeStruct((N, 2 * ATTN), jnp.float32),
        ],
    )(x2d, wq, bq, wkv, bkv)


_NC = 2
_NS = 16
_NW = _NC * _NS
_E_TOT = N * K
_NPAIR = _E_TOT // 2
_CHUNK = 200


def _make_gather_body(p_per_w):
    nsteps = p_per_w // _CHUNK

    def _gather_body(kvtab, idxe_hbm, idxo_hbm, kvg_hbm,
                     idx_ve0, idx_vo0, idx_ve1, idx_vo1,
                     buf_e0, buf_o0, buf_e1, buf_o1, sem0, sem1):
        wid = lax.axis_index("s") * _NC + lax.axis_index("c")
        base = wid * p_per_w
        idx_v = ((idx_ve0, idx_vo0), (idx_ve1, idx_vo1))
        bufs = ((buf_e0, buf_o0), (buf_e1, buf_o1))
        sems = (sem0, sem1)

        def load_and_fire(i):
            s = i % 2
            off = base + i * _CHUNK
            pltpu.sync_copy(idxe_hbm.at[pl.ds(off, _CHUNK)], idx_v[s][0])
            pltpu.sync_copy(idxo_hbm.at[pl.ds(off, _CHUNK)], idx_v[s][1])
            ce = pltpu.async_copy(kvtab.at[idx_v[s][0]], bufs[s][0], sems[s])
            co = pltpu.async_copy(kvtab.at[idx_v[s][1]], bufs[s][1], sems[s])
            return ce, co

        def drain(i, descs):
            s = i % 2
            off = base + i * _CHUNK
            descs[0].wait()
            descs[1].wait()
            pltpu.sync_copy(bufs[s][0], kvg_hbm.at[pl.ds(off, _CHUNK), pl.ds(0, 2 * ATTN)])
            pltpu.sync_copy(bufs[s][1], kvg_hbm.at[pl.ds(off, _CHUNK), pl.ds(2 * ATTN, 2 * ATTN)])

        inflight = load_and_fire(0)
        for i in range(nsteps):
            nxt = load_and_fire(i + 1) if i + 1 < nsteps else None
            drain(i, inflight)
            inflight = nxt

    return _gather_body


def _sc_gather(kvtab, idx_e, idx_o):
    npair = idx_e.shape[0]
    mesh = plsc.VectorSubcoreMesh(core_axis_name="c", subcore_axis_name="s",
                                  num_cores=_NC, num_subcores=_NS)
    fn = pl.kernel(
        _make_gather_body(npair // _NW),
        out_type=jax.ShapeDtypeStruct((npair, DIM), jnp.float32),
        mesh=mesh,
        compiler_params=pltpu.CompilerParams(use_tc_tiling_on_sc=False),
        scratch_types=[
            pltpu.VMEM((_CHUNK,), jnp.int32),
            pltpu.VMEM((_CHUNK,), jnp.int32),
            pltpu.VMEM((_CHUNK,), jnp.int32),
            pltpu.VMEM((_CHUNK,), jnp.int32),
            pltpu.VMEM((_CHUNK, 2 * ATTN), jnp.float32),
            pltpu.VMEM((_CHUNK, 2 * ATTN), jnp.float32),
            pltpu.VMEM((_CHUNK, 2 * ATTN), jnp.float32),
            pltpu.VMEM((_CHUNK, 2 * ATTN), jnp.float32),
            pltpu.SemaphoreType.DMA,
            pltpu.SemaphoreType.DMA,
        ],
    )
    return fn(kvtab, idx_e, idx_o)


_P3 = 400
_R3 = _P3 * K // 2
_RPP = K // 2
_HALVES = (1200, 3200, 3200, 2400)


def _block_body(x_ref, q_ref, kvg_ref, rp_ref,
                w1c, b1c, w2c, b2c, tq, wsm1, bsm1, msc, bsc, maskv,
                wo2, bo, ln2g, ln2b, f1w, f1b, f2w, f2b,
                out_ref):
    dn = (((0,), (0,)), ((), ()))
    w1 = w1c[...]
    hid = jnp.concatenate(
        [lax.dot_general(rp_ref[0, j], w1, dn, preferred_element_type=jnp.float32)[None]
         for j in range(_RPP)], axis=0)
    hid = jnp.maximum(hid.reshape(_R3, DIM) + b1c[...], 0.0)
    rarv = jnp.dot(hid, w2c[...], preferred_element_type=jnp.float32) + b2c[...]

    qrow = jnp.dot(q_ref[...], tq[...], preferred_element_type=jnp.float32)
    qe = jnp.broadcast_to(qrow[None], (_RPP, _P3, DIM)).reshape(_R3, DIM)

    kvg = kvg_ref[...].reshape(_R3, DIM)
    u = jnp.tanh(qe - kvg + rarv)
    s1 = jnp.maximum(jnp.dot(u, wsm1[...], preferred_element_type=jnp.float32) + bsm1[...], 0.0)
    sc = jnp.dot(s1, msc[...], preferred_element_type=jnp.float32) + bsc[...]
    sc3 = sc.reshape(_RPP, _P3, DIM)
    mask = maskv[...]

    m = jnp.max(sc3, axis=0)
    mm = jnp.maximum(m, pltpu.roll(m, 64, axis=1))
    e = jnp.exp(sc3 - mm[None]) * mask[None]
    d = jnp.sum(e, axis=0)
    d2 = d + pltpu.roll(d, 64, axis=1) + (1.0 - mask)
    attn = e / d2[None]

    vpr = (kvg + rarv).reshape(_RPP, _P3, DIM)
    vsum = jnp.sum(attn * vpr, axis=0)

    x2 = x_ref[...] + jnp.dot(vsum, wo2[...], preferred_element_type=jnp.float32) + bo[...]

    mu = jnp.mean(x2, axis=-1, keepdims=True)
    xc = x2 - mu
    var = jnp.mean(xc * xc, axis=-1, keepdims=True)
    h2 = xc * lax.rsqrt(var + 1e-5) * ln2g[...] + ln2b[...]

    f1 = jnp.dot(h2, f1w[...], preferred_element_type=jnp.float32) + f1b[...]
    g1 = f1 * 0.5 * (1.0 + lax.erf(f1 * (2.0 ** -0.5)))
    out_ref[...] = x2 + jnp.dot(g1, f2w[...], preferred_element_type=jnp.float32) + f2b[...]


def _attn_ffn(x2d, q_all, kvg, rp6, consts, nh, boff):
    def fixed(shape):
        nd = len(shape)
        return pl.BlockSpec(shape, lambda i, _nd=nd: (0,) * _nd)

    in_specs = [
        pl.BlockSpec((_P3, DIM), lambda i: (i + boff, 0)),
        pl.BlockSpec((_P3, ATTN), lambda i: (i + boff, 0)),
        pl.BlockSpec((_RPP, _P3, DIM), lambda i: (0, i, 0)),
        pl.BlockSpec((1, _RPP, 6, _P3), lambda i: (i, 0, 0, 0)),
    ] + [fixed(c.shape) for c in consts]
    return pl.pallas_call(
        _block_body,
        grid=(nh // _P3,),
        in_specs=in_specs,
        out_specs=pl.BlockSpec((_P3, DIM), lambda i: (i, 0)),
        out_shape=jax.ShapeDtypeStruct((nh, DIM), jnp.float32),
    )(x2d, q_all, kvg, rp6, *consts)


def kernel(x, knn_idx, knn_rel_pos, ln1_g, ln1_b, ln2_g, ln2_b, Wq, bq, Wk, bk,
           Wv, bv, Wo, bo, pa1_W, pa1_b, pa2_W, pa2_b, pv1_W, pv1_b, pv2_W,
           pv2_b, sm1_W, sm1_b, sm2_W, sm2_b, ffn1_W, ffn1_b, ffn2_W, ffn2_b):
    f32 = jnp.float32
    x2d = x[0]
    wq = ln1_g[:, None] * Wq
    bq2 = (bq + ln1_b @ Wq)[None, :]
    wkv_raw = jnp.concatenate([Wk, Wv], axis=1)
    wkv = ln1_g[:, None] * wkv_raw
    bkv2 = (jnp.concatenate([bk, bv]) + ln1_b @ wkv_raw)[None, :]
    q_all, kv_all = _qkv(x2d, wq, bq2, wkv, bkv2)

    idx3 = knn_idx.reshape(N, K // 2, 2).astype(jnp.int32)
    rp_t = jnp.transpose(knn_rel_pos, (2, 1, 0))
    halves = []
    p0 = 0
    for nh in _HALVES:
        idx_e = idx3[p0:p0 + nh, :, 0].transpose(1, 0).reshape(-1)
        idx_o = idx3[p0:p0 + nh, :, 1].transpose(1, 0).reshape(-1)
        nb = nh // _P3
        rp6 = (rp_t[:, :, p0:p0 + nh].reshape(3, K // 2, 2, nb, _P3)
               .transpose(3, 1, 2, 0, 4).reshape(nb, K // 2, 6, _P3))
        halves.append((nh, p0 // _P3, idx_e, idx_o, rp6))
        p0 += nh
    kvgs = [_sc_gather(kv_all, h[2], h[3]).reshape(K // 2, h[0], DIM)
            for h in halves]

    Z = jnp.zeros((ATTN, ATTN), f32)
    inv = 1.0 / jnp.sqrt(jnp.float32(ATTN))
    sm2r = jnp.broadcast_to(sm2_W * inv, (ATTN, ATTN))

    def four(b00, b01, b10, b11, b20, b21, b30, b31):
        top = jnp.concatenate([b00, b01, b10, b11], axis=1)
        bot = jnp.concatenate([b20, b21, b30, b31], axis=1)
        return top, bot

    w1c_top = jnp.concatenate([pa1_W, pv1_W, jnp.zeros((3, 2 * ATTN), f32)], axis=1)
    w1c_bot = jnp.concatenate([jnp.zeros((3, 2 * ATTN), f32), pa1_W, pv1_W], axis=1)
    w1c = jnp.concatenate([w1c_top, w1c_bot], axis=0)
    b1c = jnp.tile(jnp.concatenate([pa1_b, pv1_b]), 2)[None, :]

    r0, r1 = four(pa2_W, Z, Z, Z, Z, pv2_W, Z, Z)
    r2, r3 = four(Z, Z, pa2_W, Z, Z, Z, Z, pv2_W)
    w2c = jnp.concatenate([r0, r1, r2, r3], axis=0)
    b2c = jnp.tile(jnp.concatenate([pa2_b, pv2_b]), 2)[None, :]

    I = jnp.eye(ATTN, dtype=f32)
    tq = jnp.concatenate([I, Z, I, Z], axis=1)

    r0, r1 = four(sm1_W, Z, Z, Z, Z, Z, Z, Z)
    r2, r3 = four(Z, Z, sm1_W, Z, Z, Z, Z, Z)
    wsm1 = jnp.concatenate([r0, r1, r2, r3], axis=0)
    bsm1 = jnp.tile(jnp.concatenate([sm1_b, jnp.zeros((ATTN,), f32)]), 2)[None, :]

    r0, r1 = four(Z, sm2r, Z, Z, Z, Z, Z, Z)
    r2, r3 = four(Z, Z, Z, Z, Z, Z, Z, sm2r)
    msc = jnp.concatenate([r0, r1, r2, r3], axis=0)
    zb = jnp.zeros((ATTN,), f32)
    bsc = jnp.tile(jnp.concatenate([zb, jnp.full((ATTN,), sm2_b[0] * inv, f32)]), 2)[None, :]

    maskv = jnp.tile(jnp.concatenate([zb, jnp.ones((ATTN,), f32)]), 2)[None, :]
    wo2 = jnp.concatenate([jnp.zeros((ATTN, DIM), f32), Wo] * 2, axis=0)

    consts = [
        w1c, b1c, w2c, b2c, tq, wsm1, bsm1, msc, bsc, maskv,
        wo2, bo[None, :], ln2_g[None, :], ln2_b[None, :],
        ffn1_W, ffn1_b[None, :], ffn2_W, ffn2_b[None, :],
    ]
    outs = [_attn_ffn(x2d, q_all, kvg, h[4], consts, h[0], h[1])
            for h, kvg in zip(halves, kvgs)]
    return jnp.concatenate(outs, axis=0)[None]


# speedup vs baseline: 1.9245x; 1.0524x over previous
import jax
import jax.numpy as jnp
from jax import lax
from jax.experimental import pallas as pl
from jax.experimental.pallas import tpu as pltpu
import jax.experimental.pallas.tpu_sc as plsc

N = 10000
K = 32
DIM = 128
ATTN = 32
VAL = 32


_P1 = 2000


def _qkv_body(x_ref, wq_ref, bq_ref, wkv_ref, bkv_ref, q_ref, kv_ref):
    x = x_ref[...]
    mu = jnp.mean(x, axis=-1, keepdims=True)
    xc = x - mu
    var = jnp.mean(xc * xc, axis=-1, keepdims=True)
    h = xc * lax.rsqrt(var + 1e-5)
    q_ref[...] = jnp.dot(h, wq_ref[...], preferred_element_type=jnp.float32) + bq_ref[...]
    kv_ref[...] = jnp.dot(h, wkv_ref[...], preferred_element_type=jnp.float32) + bkv_ref[...]


def _qkv(x2d, wq, bq, wkv, bkv):
    return pl.pallas_call(
        _qkv_body,
        grid=(N // _P1,),
        in_specs=[
            pl.BlockSpec((_P1, DIM), lambda i: (i, 0)),
            pl.BlockSpec((DIM, ATTN), lambda i: (0, 0)),
            pl.BlockSpec((1, ATTN), lambda i: (0, 0)),
            pl.BlockSpec((DIM, 2 * ATTN), lambda i: (0, 0)),
            pl.BlockSpec((1, 2 * ATTN), lambda i: (0, 0)),
        ],
        out_specs=[
            pl.BlockSpec((_P1, ATTN), lambda i: (i, 0)),
            pl.BlockSpec((_P1, 2 * ATTN), lambda i: (i, 0)),
        ],
        out_shape=[
            jax.ShapeDtypeStruct((N, ATTN), jnp.float32),
            jax.ShapeDtypeStruct((N, 2 * ATTN), jnp.float32),
        ],
    )(x2d, wq, bq, wkv, bkv)


_NC = 2
_NS = 16
_NW = _NC * _NS
_E_TOT = N * K
_NPAIR = _E_TOT // 2
_CHUNK = 200


def _make_gather_body(p_per_w):
    nsteps = p_per_w // _CHUNK

    def _gather_body(kvtab, idxe_hbm, idxo_hbm, kvg_hbm,
                     idx_ve0, idx_vo0, idx_ve1, idx_vo1,
                     buf_e0, buf_o0, buf_e1, buf_o1, sem0, sem1):
        wid = lax.axis_index("s") * _NC + lax.axis_index("c")
        base = wid * p_per_w
        idx_v = ((idx_ve0, idx_vo0), (idx_ve1, idx_vo1))
        bufs = ((buf_e0, buf_o0), (buf_e1, buf_o1))
        sems = (sem0, sem1)

        def load_and_fire(i):
            s = i % 2
            off = base + i * _CHUNK
            pltpu.sync_copy(idxe_hbm.at[pl.ds(off, _CHUNK)], idx_v[s][0])
            pltpu.sync_copy(idxo_hbm.at[pl.ds(off, _CHUNK)], idx_v[s][1])
            ce = pltpu.async_copy(kvtab.at[idx_v[s][0]], bufs[s][0], sems[s])
            co = pltpu.async_copy(kvtab.at[idx_v[s][1]], bufs[s][1], sems[s])
            return ce, co

        def drain(i, descs):
            s = i % 2
            off = base + i * _CHUNK
            descs[0].wait()
            descs[1].wait()
            pltpu.sync_copy(bufs[s][0], kvg_hbm.at[pl.ds(off, _CHUNK), pl.ds(0, 2 * ATTN)])
            pltpu.sync_copy(bufs[s][1], kvg_hbm.at[pl.ds(off, _CHUNK), pl.ds(2 * ATTN, 2 * ATTN)])

        inflight = load_and_fire(0)
        for i in range(nsteps):
            nxt = load_and_fire(i + 1) if i + 1 < nsteps else None
            drain(i, inflight)
            inflight = nxt

    return _gather_body


def _sc_gather(kvtab, idx_e, idx_o):
    npair = idx_e.shape[0]
    mesh = plsc.VectorSubcoreMesh(core_axis_name="c", subcore_axis_name="s",
                                  num_cores=_NC, num_subcores=_NS)
    fn = pl.kernel(
        _make_gather_body(npair // _NW),
        out_type=jax.ShapeDtypeStruct((npair, DIM), jnp.float32),
        mesh=mesh,
        compiler_params=pltpu.CompilerParams(use_tc_tiling_on_sc=False),
        scratch_types=[
            pltpu.VMEM((_CHUNK,), jnp.int32),
            pltpu.VMEM((_CHUNK,), jnp.int32),
            pltpu.VMEM((_CHUNK,), jnp.int32),
            pltpu.VMEM((_CHUNK,), jnp.int32),
            pltpu.VMEM((_CHUNK, 2 * ATTN), jnp.float32),
            pltpu.VMEM((_CHUNK, 2 * ATTN), jnp.float32),
            pltpu.VMEM((_CHUNK, 2 * ATTN), jnp.float32),
            pltpu.VMEM((_CHUNK, 2 * ATTN), jnp.float32),
            pltpu.SemaphoreType.DMA,
            pltpu.SemaphoreType.DMA,
        ],
    )
    return fn(kvtab, idx_e, idx_o)


_P3 = 400
_R3 = _P3 * K // 2
_RPP = K // 2
_HALVES = (5200, 4800)


def _block_body(x_ref, q_ref, kvg_ref, rp_ref,
                w1c, b1c, w2c, b2c, tq, wsm1, bsm1, msc, bsc, maskv,
                wo2, bo, ln2g, ln2b, f1w, f1b, f2w, f2b,
                out_ref):
    dn = (((0,), (0,)), ((), ()))
    w1 = w1c[...]
    hid = jnp.concatenate(
        [lax.dot_general(rp_ref[0, j], w1, dn, preferred_element_type=jnp.float32)[None]
         for j in range(_RPP)], axis=0)
    hid = jnp.maximum(hid.reshape(_R3, DIM) + b1c[...], 0.0)
    rarv = jnp.dot(hid, w2c[...], preferred_element_type=jnp.float32) + b2c[...]

    qrow = jnp.dot(q_ref[...], tq[...], preferred_element_type=jnp.float32)
    qe = jnp.broadcast_to(qrow[None], (_RPP, _P3, DIM)).reshape(_R3, DIM)

    kvg = kvg_ref[...].reshape(_R3, DIM)
    u = jnp.tanh(qe - kvg + rarv)
    s1 = jnp.maximum(jnp.dot(u, wsm1[...], preferred_element_type=jnp.float32) + bsm1[...], 0.0)
    sc = jnp.dot(s1, msc[...], preferred_element_type=jnp.float32) + bsc[...]
    sc3 = sc.reshape(_RPP, _P3, DIM)
    mask = maskv[...]

    m = jnp.max(sc3, axis=0)
    mm = jnp.maximum(m, pltpu.roll(m, 64, axis=1))
    e = jnp.exp(sc3 - mm[None]) * mask[None]
    d = jnp.sum(e, axis=0)
    d2 = d + pltpu.roll(d, 64, axis=1) + (1.0 - mask)

    vpr = (kvg + rarv).reshape(_RPP, _P3, DIM)
    vsum = jnp.sum(e * vpr, axis=0) / d2

    x2 = x_ref[...] + jnp.dot(vsum, wo2[...], preferred_element_type=jnp.float32) + bo[...]

    mu = jnp.mean(x2, axis=-1, keepdims=True)
    xc = x2 - mu
    var = jnp.mean(xc * xc, axis=-1, keepdims=True)
    h2 = xc * lax.rsqrt(var + 1e-5) * ln2g[...] + ln2b[...]

    f1 = jnp.dot(h2, f1w[...], preferred_element_type=jnp.float32) + f1b[...]
    g1 = f1 * 0.5 * (1.0 + lax.erf(f1 * (2.0 ** -0.5)))
    out_ref[...] = x2 + jnp.dot(g1, f2w[...], preferred_element_type=jnp.float32) + f2b[...]


def _attn_ffn(x2d, q_all, kvg, rp6, consts, nh, boff):
    def fixed(shape):
        nd = len(shape)
        return pl.BlockSpec(shape, lambda i, _nd=nd: (0,) * _nd)

    in_specs = [
        pl.BlockSpec((_P3, DIM), lambda i: (i + boff, 0)),
        pl.BlockSpec((_P3, ATTN), lambda i: (i + boff, 0)),
        pl.BlockSpec((_RPP, _P3, DIM), lambda i: (0, i, 0)),
        pl.BlockSpec((1, _RPP, 6, _P3), lambda i: (i, 0, 0, 0)),
    ] + [fixed(c.shape) for c in consts]
    return pl.pallas_call(
        _block_body,
        grid=(nh // _P3,),
        in_specs=in_specs,
        out_specs=pl.BlockSpec((_P3, DIM), lambda i: (i, 0)),
        out_shape=jax.ShapeDtypeStruct((nh, DIM), jnp.float32),
    )(x2d, q_all, kvg, rp6, *consts)


def kernel(x, knn_idx, knn_rel_pos, ln1_g, ln1_b, ln2_g, ln2_b, Wq, bq, Wk, bk,
           Wv, bv, Wo, bo, pa1_W, pa1_b, pa2_W, pa2_b, pv1_W, pv1_b, pv2_W,
           pv2_b, sm1_W, sm1_b, sm2_W, sm2_b, ffn1_W, ffn1_b, ffn2_W, ffn2_b):
    f32 = jnp.float32
    x2d = x[0]
    wq = ln1_g[:, None] * Wq
    bq2 = (bq + ln1_b @ Wq)[None, :]
    wkv_raw = jnp.concatenate([Wk, Wv], axis=1)
    wkv = ln1_g[:, None] * wkv_raw
    bkv2 = (jnp.concatenate([bk, bv]) + ln1_b @ wkv_raw)[None, :]
    q_all, kv_all = _qkv(x2d, wq, bq2, wkv, bkv2)

    idx3 = knn_idx.reshape(N, K // 2, 2).astype(jnp.int32)
    rp_t = jnp.transpose(knn_rel_pos, (2, 1, 0))
    halves = []
    p0 = 0
    for nh in _HALVES:
        idx_e = idx3[p0:p0 + nh, :, 0].transpose(1, 0).reshape(-1)
        idx_o = idx3[p0:p0 + nh, :, 1].transpose(1, 0).reshape(-1)
        nb = nh // _P3
        rp6 = (rp_t[:, :, p0:p0 + nh].reshape(3, K // 2, 2, nb, _P3)
               .transpose(3, 1, 2, 0, 4).reshape(nb, K // 2, 6, _P3))
        halves.append((nh, p0 // _P3, idx_e, idx_o, rp6))
        p0 += nh
    kvgs = [_sc_gather(kv_all, h[2], h[3]).reshape(K // 2, h[0], DIM)
            for h in halves]

    Z = jnp.zeros((ATTN, ATTN), f32)
    inv = 1.0 / jnp.sqrt(jnp.float32(ATTN))
    sm2r = jnp.broadcast_to(sm2_W * inv, (ATTN, ATTN))

    def four(b00, b01, b10, b11, b20, b21, b30, b31):
        top = jnp.concatenate([b00, b01, b10, b11], axis=1)
        bot = jnp.concatenate([b20, b21, b30, b31], axis=1)
        return top, bot

    w1c_top = jnp.concatenate([pa1_W, pv1_W, jnp.zeros((3, 2 * ATTN), f32)], axis=1)
    w1c_bot = jnp.concatenate([jnp.zeros((3, 2 * ATTN), f32), pa1_W, pv1_W], axis=1)
    w1c = jnp.concatenate([w1c_top, w1c_bot], axis=0)
    b1c = jnp.tile(jnp.concatenate([pa1_b, pv1_b]), 2)[None, :]

    r0, r1 = four(pa2_W, Z, Z, Z, Z, pv2_W, Z, Z)
    r2, r3 = four(Z, Z, pa2_W, Z, Z, Z, Z, pv2_W)
    w2c = jnp.concatenate([r0, r1, r2, r3], axis=0)
    b2c = jnp.tile(jnp.concatenate([pa2_b, pv2_b]), 2)[None, :]

    I = jnp.eye(ATTN, dtype=f32)
    tq = jnp.concatenate([I, Z, I, Z], axis=1)

    r0, r1 = four(sm1_W, Z, Z, Z, Z, Z, Z, Z)
    r2, r3 = four(Z, Z, sm1_W, Z, Z, Z, Z, Z)
    wsm1 = jnp.concatenate([r0, r1, r2, r3], axis=0)
    bsm1 = jnp.tile(jnp.concatenate([sm1_b, jnp.zeros((ATTN,), f32)]), 2)[None, :]

    r0, r1 = four(Z, sm2r, Z, Z, Z, Z, Z, Z)
    r2, r3 = four(Z, Z, Z, Z, Z, Z, Z, sm2r)
    msc = jnp.concatenate([r0, r1, r2, r3], axis=0)
    zb = jnp.zeros((ATTN,), f32)
    bsc = jnp.tile(jnp.concatenate([zb, jnp.full((ATTN,), sm2_b[0] * inv, f32)]), 2)[None, :]

    maskv = jnp.tile(jnp.concatenate([zb, jnp.ones((ATTN,), f32)]), 2)[None, :]
    wo2 = jnp.concatenate([jnp.zeros((ATTN, DIM), f32), Wo] * 2, axis=0)

    consts = [
        w1c, b1c, w2c, b2c, tq, wsm1, bsm1, msc, bsc, maskv,
        wo2, bo[None, :], ln2_g[None, :], ln2_b[None, :],
        ffn1_W, ffn1_b[None, :], ffn2_W, ffn2_b[None, :],
    ]
    outs = [_attn_ffn(x2d, q_all, kvg, h[4], consts, h[0], h[1])
            for h, kvg in zip(halves, kvgs)]
    return jnp.concatenate(outs, axis=0)[None]
